# baseline (speedup 1.0000x reference)
import jax
import jax.numpy as jnp
from jax import lax
from jax.experimental import pallas as pl
from jax.experimental.pallas import tpu as pltpu

HID = 16
EMB = 16
T = 5
START = 3
STOP = 4
NEG = -10000.0

_R_WIHF = 0
_R_WIHB = 64
_R_WHHF = 128
_R_WHHB = 192
_R_BF = 256
_R_BB = 260
_R_W2T = 264
_R_BT = 274
_R_TRANS = 275
_R_H0 = 280
_R_C0 = 282
_R_EMBS = 284
_R_TOTAL = 348


def _bilstm_crf_fused(
    wpk,
    score_ref, path_ref,
    xf_ref, xb_ref, whh_ref, hst_ref, hrev_ref, bpc_ref, bpr_ref, term_ref,
    sm_bpc, sm_bpr, sm_term, sems,
):
    S = path_ref.shape[1]
    f32 = jnp.float32

    z16 = jnp.zeros((HID, HID), f32)
    z96w = jnp.zeros((96, EMB), f32)
    z96r = jnp.zeros((96, 2 * HID), f32)

    xrows_f, xrows_b, rrows = [], [], []
    for g in range(4):
        blk_f = wpk[_R_WIHF + 16 * g:_R_WIHF + 16 * g + 16, :]
        blk_b = wpk[_R_WIHB + 16 * g:_R_WIHB + 16 * g + 16, :]
        hf = wpk[_R_WHHF + 16 * g:_R_WHHF + 16 * g + 16, :]
        hb = wpk[_R_WHHB + 16 * g:_R_WHHB + 16 * g + 16, :]
        xrows_f += [blk_f, z16, z96w]
        xrows_b += [z16, blk_b, z96w]
        rrows += [jnp.concatenate([hf, z16], axis=1),
                  jnp.concatenate([z16, hb], axis=1),
                  z96r]
    wx_f = jnp.concatenate(xrows_f, axis=0)
    wx_b = jnp.concatenate(xrows_b, axis=0)
    whh_ref[...] = jnp.concatenate(rrows, axis=0).T

    z1_96 = jnp.zeros((1, 96), f32)
    bias = jnp.concatenate(
        [x for g in range(4)
         for x in (wpk[_R_BF + g:_R_BF + g + 1, :],
                   wpk[_R_BB + g:_R_BB + g + 1, :], z1_96)],
        axis=1)

    w2t_f = wpk[_R_W2T + 0:_R_W2T + 5, :]
    w2t_b = wpk[_R_W2T + 5:_R_W2T + 10, :]
    zt16 = jnp.zeros((T, HID), f32)
    a2 = jnp.concatenate([w2t_f, zt16], axis=1)
    b2 = jnp.concatenate([zt16, w2t_b], axis=1)
    bt = wpk[_R_BT:_R_BT + 1, 0:T]
    trans = wpk[_R_TRANS:_R_TRANS + 5, 0:T]

    emb = wpk[_R_EMBS:_R_EMBS + 64, :]
    dn = (((1,), (1,)), ((), ()))
    xf_ref[...] = lax.dot_general(emb, wx_f, dn,
                                  preferred_element_type=f32) + bias
    xb_ref[...] = lax.dot_general(emb, wx_b, dn,
                                  preferred_element_type=f32)

    whh_t = whh_ref[...]
    h = jnp.concatenate([wpk[_R_H0:_R_H0 + 1, :],
                         wpk[_R_H0 + 1:_R_H0 + 2, :]], axis=1)
    c_st = jnp.concatenate([wpk[_R_C0:_R_C0 + 1, :],
                            wpk[_R_C0 + 1:_R_C0 + 2, :]], axis=1)

    for k in range(S):
        kr = S - 1 - k
        x = xf_ref[pl.ds(k, 1), :] + xb_ref[pl.ds(kr, 1), :]
        m = x + jnp.dot(h, whh_t, preferred_element_type=f32)
        si = jax.nn.sigmoid(m[:, 0:32])
        sf = jax.nn.sigmoid(m[:, 128:160])
        tg = jnp.tanh(m[:, 256:288])
        so = jax.nn.sigmoid(m[:, 384:416])
        c_st = sf * c_st + si * tg
        h = so * jnp.tanh(c_st)
        hst_ref[pl.ds(k, 1), :] = h
        hrev_ref[pl.ds(kr, 1), :] = h

    feats = (lax.dot_general(hst_ref[...], a2, dn, preferred_element_type=f32)
             + lax.dot_general(hrev_ref[...], b2, dn, preferred_element_type=f32)
             + bt)
    ft_t = feats.T

    lane_t = lax.broadcasted_iota(jnp.int32, (1, T), 1)
    lane2 = lax.broadcasted_iota(jnp.int32, (T, T), 1)
    sub2 = lax.broadcasted_iota(jnp.int32, (T, T), 0)
    trans_t = trans.T
    z55 = jnp.zeros((T, T), f32)
    ftreps = [ft_t[:, t:t + 1] + z55 for t in range(0, S, 2)]
    frows = [feats[t:t + 1, :] for t in range(1, S, 2)]

    fv_row = jnp.where(lane_t == START, 0.0, NEG)
    fv_col = None
    for t in range(S):
        if t % 2 == 0:
            nvar = trans + fv_row
            best = jnp.max(nvar, axis=1, keepdims=True)
            bp = jnp.min(jnp.where(nvar == best, lane2, T), axis=1,
                         keepdims=True)
            bpc_ref[:, t:t + 1] = bp
            fv_col = best + ftreps[t // 2]
        else:
            nvar = trans_t + fv_col
            best = jnp.max(nvar, axis=0, keepdims=True)
            bp = jnp.min(jnp.where(nvar == best, sub2, T), axis=0,
                         keepdims=True)
            bpr_ref[pl.ds(t, 1), :] = bp
            fv_row = best + frows[t // 2]

    cp_c = pltpu.make_async_copy(bpc_ref, sm_bpc, sems.at[0])
    cp_r = pltpu.make_async_copy(bpr_ref, sm_bpr, sems.at[1])
    cp_c.start()
    cp_r.start()

    terminal = fv_row + trans[STOP:STOP + 1, :]
    score_ref[...] = jnp.max(terminal, axis=1, keepdims=True)
    term_ref[...] = terminal
    cp_t = pltpu.make_async_copy(term_ref, sm_term, sems.at[2])
    cp_t.start()

    cp_c.wait()
    cp_r.wait()
    cp_t.wait()

    best_v = sm_term[0, 0]
    best_i = jnp.int32(0)
    for j in range(1, T):
        better = sm_term[0, j] > best_v
        best_v = jnp.where(better, sm_term[0, j], best_v)
        best_i = jnp.where(better, jnp.int32(j), best_i)

    iota_s = lax.broadcasted_iota(jnp.int32, (1, S), 1)
    cur = best_i
    path_row = jnp.where(iota_s == (S - 1), cur, 0)
    for k in range(S - 1):
        t = S - 1 - k
        if t % 2 == 0:
            prev = sm_bpc[cur, t]
        else:
            prev = sm_bpr[t, cur]
        path_row = jnp.where(iota_s == (t - 1), prev, path_row)
        cur = prev
    path_ref[...] = path_row


def kernel(sentence, embedding, w_ih_f, w_hh_f, b_ih_f, b_hh_f,
           w_ih_b, w_hh_b, b_ih_b, b_hh_b, w_h2t, b_h2t, transitions, h0, c0):
    S = sentence.shape[0]
    f32 = jnp.float32

    packed = jnp.concatenate([
        w_ih_f, w_ih_b, w_hh_f, w_hh_b,
        (b_ih_f + b_hh_f).reshape(4, HID),
        (b_ih_b + b_hh_b).reshape(4, HID),
        w_h2t[:, :HID], w_h2t[:, HID:],
        jnp.pad(b_h2t.reshape(1, T), ((0, 0), (0, HID - T))),
        jnp.pad(transitions, ((0, 0), (0, HID - T))),
        h0.reshape(2, HID), c0.reshape(2, HID),
        embedding[sentence],
    ], axis=0)

    def _vmem_spec(shape):
        nd = len(shape)
        return pl.BlockSpec(shape, lambda *_, _nd=nd: (0,) * _nd)

    score, path = pl.pallas_call(
        _bilstm_crf_fused,
        out_shape=(jax.ShapeDtypeStruct((1, 1), f32),
                   jax.ShapeDtypeStruct((1, S), jnp.int32)),
        grid_spec=pltpu.PrefetchScalarGridSpec(
            num_scalar_prefetch=0,
            grid=(1,),
            in_specs=[_vmem_spec(packed.shape)],
            out_specs=[_vmem_spec((1, 1)), _vmem_spec((1, S))],
            scratch_shapes=[
                pltpu.VMEM((S, 512), f32),
                pltpu.VMEM((S, 512), f32),
                pltpu.VMEM((32, 512), f32),
                pltpu.VMEM((S, 2 * HID), f32),
                pltpu.VMEM((S, 2 * HID), f32),
                pltpu.VMEM((T, S), jnp.int32),
                pltpu.VMEM((S, T), jnp.int32),
                pltpu.VMEM((1, T), f32),
                pltpu.SMEM((T, S), jnp.int32),
                pltpu.SMEM((S, T), jnp.int32),
                pltpu.SMEM((1, T), f32),
                pltpu.SemaphoreType.DMA((3,)),
            ]),
        compiler_params=pltpu.CompilerParams(
            dimension_semantics=("arbitrary",)),
    )(packed)
    return score[0, 0], path[0, :]


# speedup vs baseline: 1.1475x; 1.1475x over previous
import jax
import jax.numpy as jnp
from jax import lax
from jax.experimental import pallas as pl
from jax.experimental.pallas import tpu as pltpu

HID = 16
EMB = 16
T = 5
START = 3
STOP = 4
NEG = -10000.0

_R_WIHF = 0
_R_WIHB = 64
_R_WHHF = 128
_R_WHHB = 192
_R_BF = 256
_R_BB = 260
_R_W2T = 264
_R_BT = 274
_R_TRANS = 275
_R_H0 = 280
_R_C0 = 282
_R_TOTAL = 284


def _bilstm_crf_fused(
    embs, wpk,
    score_ref, path_ref,
    xf_ref, xb_ref, whh_ref, hst_ref, hrev_ref, bpc_ref, bpr_ref, term_ref,
    sm_bpc, sm_bpr, sm_term, sems,
):
    S = path_ref.shape[1]
    f32 = jnp.float32

    z16 = jnp.zeros((HID, HID), f32)
    z96w = jnp.zeros((96, EMB), f32)
    z96r = jnp.zeros((96, 2 * HID), f32)

    xrows_f, xrows_b, rrows = [], [], []
    for g in range(4):
        blk_f = wpk[_R_WIHF + 16 * g:_R_WIHF + 16 * g + 16, :]
        blk_b = wpk[_R_WIHB + 16 * g:_R_WIHB + 16 * g + 16, :]
        hf = wpk[_R_WHHF + 16 * g:_R_WHHF + 16 * g + 16, :]
        hb = wpk[_R_WHHB + 16 * g:_R_WHHB + 16 * g + 16, :]
        xrows_f += [blk_f, z16, z96w]
        xrows_b += [z16, blk_b, z96w]
        rrows += [jnp.concatenate([hf, z16], axis=1),
                  jnp.concatenate([z16, hb], axis=1),
                  z96r]
    wx_f = jnp.concatenate(xrows_f, axis=0)
    wx_b = jnp.concatenate(xrows_b, axis=0)
    whh_ref[...] = jnp.concatenate(rrows, axis=0).T

    z1_96 = jnp.zeros((1, 96), f32)
    bias = jnp.concatenate(
        [x for g in range(4)
         for x in (wpk[_R_BF + g:_R_BF + g + 1, :],
                   wpk[_R_BB + g:_R_BB + g + 1, :], z1_96)],
        axis=1)

    w2t_f = wpk[_R_W2T + 0:_R_W2T + 5, :]
    w2t_b = wpk[_R_W2T + 5:_R_W2T + 10, :]
    zt16 = jnp.zeros((T, HID), f32)
    a2 = jnp.concatenate([w2t_f, zt16], axis=1)
    b2 = jnp.concatenate([zt16, w2t_b], axis=1)
    bt = wpk[_R_BT:_R_BT + 1, 0:T]
    trans = wpk[_R_TRANS:_R_TRANS + 5, 0:T]

    emb = embs[...]
    dn = (((1,), (1,)), ((), ()))
    xf_ref[...] = lax.dot_general(emb, wx_f, dn,
                                  preferred_element_type=f32) + bias
    xb_ref[...] = lax.dot_general(emb, wx_b, dn,
                                  preferred_element_type=f32)

    whh_t = whh_ref[...]
    h = jnp.concatenate([wpk[_R_H0:_R_H0 + 1, :],
                         wpk[_R_H0 + 1:_R_H0 + 2, :]], axis=1)
    c_st = jnp.concatenate([wpk[_R_C0:_R_C0 + 1, :],
                            wpk[_R_C0 + 1:_R_C0 + 2, :]], axis=1)

    for k in range(S):
        kr = S - 1 - k
        x = xf_ref[pl.ds(k, 1), :] + xb_ref[pl.ds(kr, 1), :]
        m = x + jnp.dot(h, whh_t, preferred_element_type=f32)
        si = jax.nn.sigmoid(m[:, 0:32])
        sf = jax.nn.sigmoid(m[:, 128:160])
        tg = jnp.tanh(m[:, 256:288])
        so = jax.nn.sigmoid(m[:, 384:416])
        c_st = sf * c_st + si * tg
        h = so * jnp.tanh(c_st)
        hst_ref[pl.ds(k, 1), :] = h
        hrev_ref[pl.ds(kr, 1), :] = h

    feats = (lax.dot_general(hst_ref[...], a2, dn, preferred_element_type=f32)
             + lax.dot_general(hrev_ref[...], b2, dn, preferred_element_type=f32)
             + bt)
    ft_t = feats.T

    lane_t = lax.broadcasted_iota(jnp.int32, (1, T), 1)
    lane2 = lax.broadcasted_iota(jnp.int32, (T, T), 1)
    sub2 = lax.broadcasted_iota(jnp.int32, (T, T), 0)
    trans_t = trans.T
    z55 = jnp.zeros((T, T), f32)
    ftreps = [ft_t[:, t:t + 1] + z55 for t in range(0, S, 2)]
    frows = [feats[t:t + 1, :] for t in range(1, S, 2)]

    fv_row = jnp.where(lane_t == START, 0.0, NEG)
    fv_col = None
    for t in range(S):
        if t % 2 == 0:
            nvar = trans + fv_row
            best = jnp.max(nvar, axis=1, keepdims=True)
            bp = jnp.min(jnp.where(nvar == best, lane2, T), axis=1,
                         keepdims=True)
            bpc_ref[:, t:t + 1] = bp
            fv_col = best + ftreps[t // 2]
        else:
            nvar = trans_t + fv_col
            best = jnp.max(nvar, axis=0, keepdims=True)
            bp = jnp.min(jnp.where(nvar == best, sub2, T), axis=0,
                         keepdims=True)
            bpr_ref[pl.ds(t, 1), :] = bp
            fv_row = best + frows[t // 2]

    cp_c = pltpu.make_async_copy(bpc_ref, sm_bpc, sems.at[0])
    cp_r = pltpu.make_async_copy(bpr_ref, sm_bpr, sems.at[1])
    cp_c.start()
    cp_r.start()

    terminal = fv_row + trans[STOP:STOP + 1, :]
    score_ref[...] = jnp.max(terminal, axis=1, keepdims=True)
    term_ref[...] = terminal
    cp_t = pltpu.make_async_copy(term_ref, sm_term, sems.at[2])
    cp_t.start()

    cp_c.wait()
    cp_r.wait()
    cp_t.wait()

    best_v = sm_term[0, 0]
    best_i = jnp.int32(0)
    for j in range(1, T):
        better = sm_term[0, j] > best_v
        best_v = jnp.where(better, sm_term[0, j], best_v)
        best_i = jnp.where(better, jnp.int32(j), best_i)

    iota_s = lax.broadcasted_iota(jnp.int32, (1, S), 1)
    cur = best_i
    path_row = jnp.where(iota_s == (S - 1), cur, 0)
    for k in range(S - 1):
        t = S - 1 - k
        if t % 2 == 0:
            prev = sm_bpc[cur, t]
        else:
            prev = sm_bpr[t, cur]
        path_row = jnp.where(iota_s == (t - 1), prev, path_row)
        cur = prev
    path_ref[...] = path_row


def kernel(sentence, embedding, w_ih_f, w_hh_f, b_ih_f, b_hh_f,
           w_ih_b, w_hh_b, b_ih_b, b_hh_b, w_h2t, b_h2t, transitions, h0, c0):
    S = sentence.shape[0]
    f32 = jnp.float32

    embs = embedding[sentence]

    packed = jnp.concatenate([
        w_ih_f, w_ih_b, w_hh_f, w_hh_b,
        (b_ih_f + b_hh_f).reshape(4, HID),
        (b_ih_b + b_hh_b).reshape(4, HID),
        w_h2t[:, :HID], w_h2t[:, HID:],
        jnp.pad(b_h2t.reshape(1, T), ((0, 0), (0, HID - T))),
        jnp.pad(transitions, ((0, 0), (0, HID - T))),
        h0.reshape(2, HID), c0.reshape(2, HID),
    ], axis=0)

    def _vmem_spec(shape):
        nd = len(shape)
        return pl.BlockSpec(shape, lambda *_, _nd=nd: (0,) * _nd)

    score, path = pl.pallas_call(
        _bilstm_crf_fused,
        out_shape=(jax.ShapeDtypeStruct((1, 1), f32),
                   jax.ShapeDtypeStruct((1, S), jnp.int32)),
        grid_spec=pltpu.PrefetchScalarGridSpec(
            num_scalar_prefetch=0,
            grid=(1,),
            in_specs=[_vmem_spec(embs.shape), _vmem_spec(packed.shape)],
            out_specs=[_vmem_spec((1, 1)), _vmem_spec((1, S))],
            scratch_shapes=[
                pltpu.VMEM((S, 512), f32),
                pltpu.VMEM((S, 512), f32),
                pltpu.VMEM((32, 512), f32),
                pltpu.VMEM((S, 2 * HID), f32),
                pltpu.VMEM((S, 2 * HID), f32),
                pltpu.VMEM((T, S), jnp.int32),
                pltpu.VMEM((S, T), jnp.int32),
                pltpu.VMEM((1, T), f32),
                pltpu.SMEM((T, S), jnp.int32),
                pltpu.SMEM((S, T), jnp.int32),
                pltpu.SMEM((1, T), f32),
                pltpu.SemaphoreType.DMA((3,)),
            ]),
        compiler_params=pltpu.CompilerParams(
            dimension_semantics=("arbitrary",)),
    )(embs, packed)
    return score[0, 0], path[0, :]
